# Initial kernel scaffold; baseline (speedup 1.0000x reference)
import jax
import jax.numpy as jnp
from jax.experimental import pallas as pl


def kernel(X, A1, A2, A3, edge_feature, params):
    raise NotImplementedError("write your pallas kernel here")


# speedup vs baseline: 36.9869x; 36.9869x over previous
import functools

import jax
import jax.numpy as jnp
from jax import lax
from jax.experimental import pallas as pl
from jax.experimental.pallas import tpu as pltpu
from jax.experimental.pallas import tpu_sc as plsc

N = 50000
E = 800000
IN_FEATS = 32
H_FEATS = 16
HEADS = 4
HH = HEADS * H_FEATS

NC, NS, L = 2, 16, 16
ROWS_PT = N // NS

BLK = 400
GRID = N // BLK

B1 = 128
EPT1 = E // NS
C1_FULL = EPT1 // B1
R1 = EPT1 - C1_FULL * B1

B2 = 128
EPT2 = E // (NC * NS)
C2_FULL = EPT2 // B2
R2 = EPT2 - C2_FULL * B2
R2P = 48

EPS = 1e-16

DCH = 6256
DLAST = 2 * N - (NS - 1) * DCH

_SC_PARAMS = pltpu.CompilerParams(use_tc_tiling_on_sc=False)


def _copy_flat_slab(s, src_ref, dst_ref):
    @pl.when(s < NS - 1)
    def _():
        pltpu.sync_copy(src_ref.at[pl.ds(s * DCH, DCH)], dst_ref.at[pl.ds(s * DCH, DCH)])

    @pl.when(s == NS - 1)
    def _():
        pltpu.sync_copy(
            src_ref.at[pl.ds((NS - 1) * DCH, DLAST)],
            dst_ref.at[pl.ds((NS - 1) * DCH, DLAST)],
        )


def _lrelu_exp(x):
    return jnp.exp(jnp.maximum(x, 0.2 * x))


def _stage1_body(x_ref, em_ref, *refs):
    x = x_ref[...]
    for i in range(3):
        w_ref, as_ref, ad_ref = refs[3 * i], refs[3 * i + 1], refs[3 * i + 2]
        h_ref, a_ref, d_ref, ia_ref, id_ref = refs[9 + 5 * i : 9 + 5 * i + 5]
        h = jnp.dot(x, w_ref[...], preferred_element_type=jnp.float32, precision=lax.Precision.HIGHEST)
        asrc = jnp.dot(h, as_ref[...], preferred_element_type=jnp.float32, precision=lax.Precision.HIGHEST)
        adst = jnp.dot(h, ad_ref[...], preferred_element_type=jnp.float32, precision=lax.Precision.HIGHEST)
        ex0 = _lrelu_exp(asrc + adst)
        ex64 = jnp.dot(ex0, em_ref[...], preferred_element_type=jnp.float32, precision=lax.Precision.HIGHEST)
        ia = ex64 * h
        h_ref[0] = h[:, :32]
        h_ref[1] = h[:, 32:]
        a_ref[...] = asrc
        d_ref[...] = adst
        ia_ref[0] = ia[:, :32]
        ia_ref[1] = ia[:, 32:]
        id_ref[0] = ex0[:, :2]
        id_ref[1] = ex0[:, 2:]


def _stage1(X, Ws, As, Ads, Em):
    full = lambda shp: pl.BlockSpec(shp, lambda i: (0,) * len(shp))
    in_specs = [pl.BlockSpec((BLK, IN_FEATS), lambda i: (i, 0)), full((HEADS, HH))]
    for _ in range(3):
        in_specs += [full((IN_FEATS, HH)), full((HH, HEADS)), full((HH, HEADS))]
    out_specs, out_shape = [], []
    for _ in range(3):
        for shp, blk in (
            ((NC, N, 32), (NC, BLK, 32)),
            ((N, HEADS), (BLK, HEADS)),
            ((N, HEADS), (BLK, HEADS)),
            ((NC, N, 32), (NC, BLK, 32)),
            ((NC, N, 2), (NC, BLK, 2)),
        ):
            ix = (lambda i: (0, i, 0)) if len(shp) == 3 else (lambda i: (i, 0))
            out_specs.append(pl.BlockSpec(blk, ix))
            out_shape.append(jax.ShapeDtypeStruct(shp, jnp.float32))
    args = [X, Em]
    for i in range(3):
        args += [Ws[i], As[i], Ads[i]]
    return pl.pallas_call(
        _stage1_body,
        grid=(GRID,),
        in_specs=in_specs,
        out_specs=out_specs,
        out_shape=out_shape,
    )(*args)


def _edge_chunk1(src_h, dst_h, asr_h, adt_h, hsp_h, acc_sh, den_sh, bufs, sem,
                 c, eb, nb):
    (sidx, didx, sg0, sg1, dg0, dg1, hidx, dd0, dd1,
     ab0, ab1, db0, db1, ex0, ex1, hbuf, msg) = bufs
    pltpu.sync_copy(src_h.at[pl.ds(eb, nb)], sidx)
    pltpu.sync_copy(dst_h.at[pl.ds(eb, nb)], didx)
    c2 = c * 2
    for g in range(nb // L):
        sl = pl.ds(g * L, L)
        sv = sidx[sl]
        dv = didx[sl]
        s4 = sv * 4 + c2
        d4 = dv * 4 + c2
        sg0[sl] = s4
        sg1[sl] = s4 + 1
        dg0[sl] = d4
        dg1[sl] = d4 + 1
        hidx[sl] = sv + c * N
        d2 = dv * 2
        dd0[sl] = d2
        dd1[sl] = d2 + 1
    pltpu.async_copy(asr_h.at[sg0], ab0, sem).wait()
    pltpu.async_copy(asr_h.at[sg1], ab1, sem).wait()
    pltpu.async_copy(adt_h.at[dg0], db0, sem).wait()
    pltpu.async_copy(adt_h.at[dg1], db1, sem).wait()
    pltpu.async_copy(hsp_h.at[hidx], hbuf, sem).wait()
    for g in range(nb // L):
        sl = pl.ds(g * L, L)
        ex0[sl] = _lrelu_exp(ab0[sl] + db0[sl])
        ex1[sl] = _lrelu_exp(ab1[sl] + db1[sl])
    def _msg_grp(g, _):
        e0v = ex0[pl.ds(g * L, L)]
        e1v = ex1[pl.ds(g * L, L)]
        for j in range(L):
            r = g * L + j
            msg[r, pl.ds(0, L)] = jnp.full((L,), e0v[j], jnp.float32) * hbuf[r, pl.ds(0, L)]
            msg[r, pl.ds(L, L)] = jnp.full((L,), e1v[j], jnp.float32) * hbuf[r, pl.ds(L, L)]
        return 0

    lax.fori_loop(0, nb // L, _msg_grp, 0)
    pltpu.sync_copy(ex0, den_sh.at[dd0], add=True)
    pltpu.sync_copy(ex1, den_sh.at[dd1], add=True)
    pltpu.sync_copy(msg, acc_sh.at[didx], add=True)


def _mkbufs1(nb):
    return [pltpu.VMEM((nb,), jnp.int32) for _ in range(9)] + [
        pltpu.VMEM((nb,), jnp.float32) for _ in range(6)
    ] + [pltpu.VMEM((nb, 32), jnp.float32) for _ in range(2)]


def _gat1_edges(src, dst, asr, adt, hsp, iacc, iden):
    mesh = plsc.VectorSubcoreMesh(core_axis_name="c", subcore_axis_name="s")

    @functools.partial(
        pl.kernel,
        out_type=(
            jax.ShapeDtypeStruct((NC, N, 32), jnp.float32),
            jax.ShapeDtypeStruct((NC, 2 * N), jnp.float32),
        ),
        mesh=mesh,
        scratch_types=[
            pltpu.VMEM_SHARED((N, 32), jnp.float32),
            pltpu.VMEM_SHARED((2 * N,), jnp.float32),
        ]
        + _mkbufs1(B1)
        + _mkbufs1(R1)
        + [pltpu.SemaphoreType.DMA],
        compiler_params=_SC_PARAMS,
    )
    def k(src_h, dst_h, asr_h, adt_h, hsp_h, iacc_h, iden_h, acc_out, den_out,
          acc_sh, den_sh, *scr):
        bufs = scr[:17]
        bufr = scr[17:34]
        sem = scr[34]
        c = lax.axis_index("c")
        s = lax.axis_index("s")
        r0 = s * ROWS_PT
        pltpu.sync_copy(iacc_h.at[c, pl.ds(r0, ROWS_PT)], acc_sh.at[pl.ds(r0, ROWS_PT)])
        _copy_flat_slab(s, iden_h.at[c], den_sh)
        plsc.subcore_barrier()
        base = s * EPT1

        def body(j, _):
            _edge_chunk1(src_h, dst_h, asr_h, adt_h, hsp_h, acc_sh, den_sh,
                         bufs, sem, c, base + j * B1, B1)
            return 0

        lax.fori_loop(0, C1_FULL, body, 0)
        _edge_chunk1(src_h, dst_h, asr_h, adt_h, hsp_h, acc_sh, den_sh,
                     bufr, sem, c, base + C1_FULL * B1, R1)
        plsc.subcore_barrier()
        pltpu.sync_copy(acc_sh.at[pl.ds(r0, ROWS_PT)], acc_out.at[c, pl.ds(r0, ROWS_PT)])
        _copy_flat_slab(s, den_sh, den_out.at[c])

    return k(src, dst, asr, adt, hsp, iacc, iden)


def _stage2_body(e2_ref, *refs):
    xs = []
    for i in range(3):
        acc_ref, den_ref, b1_ref = refs[5 * i], refs[5 * i + 1], refs[5 * i + 2]
        parts = []
        for cc in range(NC):
            den32 = jnp.dot(den_ref[cc], e2_ref[...], preferred_element_type=jnp.float32, precision=lax.Precision.HIGHEST)
            parts.append(acc_ref[cc] / (den32 + EPS) + b1_ref[0, 32 * cc : 32 * cc + 32])
        xs.append(jax.nn.relu(jnp.concatenate(parts, axis=-1)))
    xc = jnp.concatenate(xs, axis=-1)
    for i in range(3):
        w2_ref, sc_ref = refs[5 * i + 3], refs[5 * i + 4]
        s2_ref, h2o_ref, d2_ref, it_ref = refs[15 + 4 * i : 15 + 4 * i + 4]
        h2 = jnp.dot(xc, w2_ref[...], preferred_element_type=jnp.float32, precision=lax.Precision.HIGHEST)
        a2s = sc_ref[0, 0] * h2
        a2d = sc_ref[0, 1] * h2
        e0 = _lrelu_exp(a2s + a2d)
        s2_ref[...] = a2s
        h2o_ref[...] = h2
        d2_ref[...] = a2d
        it_ref[0] = jnp.concatenate([e0, e0 * h2], axis=-1)
        it_ref[1] = jnp.zeros((h2.shape[0], 2), jnp.float32)


def _stage2(accs, dens, b1s, w2s, scs, E2):
    full = lambda shp: pl.BlockSpec(shp, lambda i: (0,) * len(shp))
    in_specs = [full((2, 32))]
    args = [E2]
    for i in range(3):
        in_specs += [
            pl.BlockSpec((NC, BLK, 32), lambda i: (0, i, 0)),
            pl.BlockSpec((NC, BLK, 2), lambda i: (0, i, 0)),
            full((1, HH)),
            full((3 * HH, 1)),
            full((1, 2)),
        ]
        args += [accs[i], dens[i], b1s[i], w2s[i], scs[i]]
    out_specs, out_shape = [], []
    for _ in range(3):
        out_specs += [
            pl.BlockSpec((BLK, 1), lambda i: (i, 0)),
            pl.BlockSpec((BLK, 1), lambda i: (i, 0)),
            pl.BlockSpec((BLK, 1), lambda i: (i, 0)),
            pl.BlockSpec((NC, BLK, 2), lambda i: (0, i, 0)),
        ]
        out_shape += [
            jax.ShapeDtypeStruct((N, 1), jnp.float32),
            jax.ShapeDtypeStruct((N, 1), jnp.float32),
            jax.ShapeDtypeStruct((N, 1), jnp.float32),
            jax.ShapeDtypeStruct((NC, N, 2), jnp.float32),
        ]
    return pl.pallas_call(
        _stage2_body,
        grid=(GRID,),
        in_specs=in_specs,
        out_specs=out_specs,
        out_shape=out_shape,
    )(*args)


def _edge_chunk2(src_h, dst_h, s2a_h, s2h_h, d2a_h, t2_sh, bufs, sem, eb, nb,
                 valid=None):
    sidx, didx, dd0, dd1, ab, hb, db, exb, nmb = bufs
    nv = nb if valid is None else valid
    pltpu.sync_copy(src_h.at[pl.ds(eb, nv)], sidx.at[pl.ds(0, nv)])
    pltpu.sync_copy(dst_h.at[pl.ds(eb, nv)], didx.at[pl.ds(0, nv)])
    iot = lax.iota(jnp.int32, L)
    for g in range(nb // L):
        sl = pl.ds(g * L, L)
        if valid is not None:
            m = (g * L + iot) < valid
            sidx[sl] = jnp.where(m, sidx[sl], 0)
            didx[sl] = jnp.where(m, didx[sl], 0)
        d2 = didx[sl] * 2
        dd0[sl] = d2
        dd1[sl] = d2 + 1
    pltpu.async_copy(s2a_h.at[sidx], ab, sem).wait()
    pltpu.async_copy(s2h_h.at[sidx], hb, sem).wait()
    pltpu.async_copy(d2a_h.at[didx], db, sem).wait()
    for g in range(nb // L):
        sl = pl.ds(g * L, L)
        ex = _lrelu_exp(ab[sl] + db[sl])
        if valid is not None:
            ex = jnp.where((g * L + iot) < valid, ex, 0.0)
        exb[sl] = ex
        nmb[sl] = ex * hb[sl]
    pltpu.sync_copy(exb, t2_sh.at[dd0], add=True)
    pltpu.sync_copy(nmb, t2_sh.at[dd1], add=True)


def _mkbufs2(nb):
    return [pltpu.VMEM((nb,), jnp.int32) for _ in range(4)] + [
        pltpu.VMEM((nb,), jnp.float32) for _ in range(5)
    ]


def _gat2_edges(srcs, dsts, s2as, s2hs, d2as, its):
    mesh = plsc.VectorSubcoreMesh(core_axis_name="c", subcore_axis_name="s")

    @functools.partial(
        pl.kernel,
        out_type=tuple(
            jax.ShapeDtypeStruct((NC, 2 * N), jnp.float32) for _ in range(3)
        ),
        mesh=mesh,
        scratch_types=[pltpu.VMEM_SHARED((2 * N,), jnp.float32) for _ in range(3)]
        + _mkbufs2(B2)
        + _mkbufs2(R2P)
        + [pltpu.SemaphoreType.DMA],
        compiler_params=_SC_PARAMS,
    )
    def k(s1, d1, s2, d2, s3, d3, a1, h1, q1, a2, h2, q2, a3, h3, q3,
          i1, i2, i3, o1, o2, o3, t1_sh, t2_sh, t3_sh, *scr):
        bufs = scr[:9]
        bufr = scr[9:18]
        sem = scr[18]
        c = lax.axis_index("c")
        s = lax.axis_index("s")
        r0 = s * ROWS_PT
        srcl, dstl = (s1, s2, s3), (d1, d2, d3)
        tabl = ((a1, h1, q1), (a2, h2, q2), (a3, h3, q3))
        itl = (i1, i2, i3)
        outl = (o1, o2, o3)
        shl = (t1_sh, t2_sh, t3_sh)
        for a in range(3):
            _copy_flat_slab(s, itl[a].at[c], shl[a])
        plsc.subcore_barrier()
        base = c * (E // NC) + s * EPT2
        for a in range(3):
            def body(j, _, a=a):
                _edge_chunk2(srcl[a], dstl[a], *tabl[a], shl[a], bufs, sem,
                             base + j * B2, B2)
                return 0

            lax.fori_loop(0, C2_FULL, body, 0)
            _edge_chunk2(srcl[a], dstl[a], *tabl[a], shl[a], bufr, sem,
                         base + C2_FULL * B2, R2P, valid=R2)
        plsc.subcore_barrier()
        for a in range(3):
            _copy_flat_slab(s, shl[a], outl[a].at[c])

    return k(srcs[0], dsts[0], srcs[1], dsts[1], srcs[2], dsts[2],
             s2as[0], s2hs[0], d2as[0], s2as[1], s2hs[1], d2as[1],
             s2as[2], s2hs[2], d2as[2], its[0], its[1], its[2])


def _final_body(t1_ref, t2_ref, t3_ref, cv_ref, out_ref):
    cv = cv_ref[...]
    acc = None
    for i, t_ref in enumerate((t1_ref, t2_ref, t3_ref)):
        t = t_ref[0] + t_ref[1]
        y = t[:, 1:2] / (t[:, 0:1] + EPS) + cv[0, i]
        term = y * cv[0, 3 + i]
        acc = term if acc is None else acc + term
    out_ref[...] = acc + cv[0, 6]


def _final(t2outs, cvec):
    blk3 = pl.BlockSpec((NC, BLK, 2), lambda i: (0, i, 0))
    return pl.pallas_call(
        _final_body,
        grid=(GRID,),
        in_specs=[blk3, blk3, blk3, pl.BlockSpec((1, 8), lambda i: (0, 0))],
        out_specs=pl.BlockSpec((BLK, 1), lambda i: (i, 0)),
        out_shape=jax.ShapeDtypeStruct((N, 1), jnp.float32),
    )(*t2outs, cvec)


def kernel(X, A1, A2, A3, edge_feature, params):
    p = params
    As, Ads, Ws, b1s, w2s, scs = [], [], [], [], [], []
    for i in (1, 2, 3):
        a_s = p[f"as1_{i}"].reshape(HEADS, H_FEATS)
        a_d = p[f"ad1_{i}"].reshape(HEADS, H_FEATS)
        zer = jnp.zeros((HEADS, HH), jnp.float32)
        rows = jnp.arange(HEADS)[:, None]
        cols = rows * H_FEATS + jnp.arange(H_FEATS)[None, :]
        As.append(zer.at[rows, cols].set(a_s).T)
        Ads.append(zer.at[rows, cols].set(a_d).T)
        Ws.append(p[f"W1_{i}"])
        b1s.append(p[f"b1_{i}"].reshape(1, HH))
        w2s.append(p[f"W2_{i}"])
        scs.append(jnp.stack([p[f"as2_{i}"].reshape(()), p[f"ad2_{i}"].reshape(())]).reshape(1, 2))
    Em = jnp.zeros((HEADS, HH), jnp.float32).at[
        jnp.arange(HEADS)[:, None],
        jnp.arange(HEADS)[:, None] * H_FEATS + jnp.arange(H_FEATS)[None, :],
    ].set(1.0)
    E2 = jnp.zeros((2, 32), jnp.float32).at[
        jnp.arange(2)[:, None],
        jnp.arange(2)[:, None] * H_FEATS + jnp.arange(H_FEATS)[None, :],
    ].set(1.0)

    s1 = _stage1(X, Ws, As, Ads, Em)
    accs, dens = [], []
    srcs, dsts = [], []
    for i, A in enumerate((A1, A2, A3)):
        H, Aq, D, IA, ID = s1[5 * i : 5 * i + 5]
        src = A[0].astype(jnp.int32)
        dst = A[1].astype(jnp.int32)
        srcs.append(src)
        dsts.append(dst)
        acc, den = _gat1_edges(
            src, dst,
            Aq.reshape(N * HEADS), D.reshape(N * HEADS), H.reshape(NC * N, 32),
            IA, ID.reshape(NC, 2 * N),
        )
        accs.append(acc)
        dens.append(den.reshape(NC, N, 2))

    s2 = _stage2(accs, dens, b1s, w2s, scs, E2)
    s2as = [s2[4 * i].reshape(N) for i in range(3)]
    s2hs = [s2[4 * i + 1].reshape(N) for i in range(3)]
    d2as = [s2[4 * i + 2].reshape(N) for i in range(3)]
    its = [s2[4 * i + 3].reshape(NC, 2 * N) for i in range(3)]

    t2outs = _gat2_edges(srcs, dsts, s2as, s2hs, d2as, its)

    cvec = jnp.concatenate(
        [
            jnp.stack([p["b2_1"][0], p["b2_2"][0], p["b2_3"][0]]),
            p["Wln"][:, 0],
            p["bln"],
            jnp.zeros((1,), jnp.float32),
        ]
    ).reshape(1, 8)
    return _final([t.reshape(NC, N, 2) for t in t2outs], cvec)


# speedup vs baseline: 61.3620x; 1.6590x over previous
import functools

import jax
import jax.numpy as jnp
from jax import lax
from jax.experimental import pallas as pl
from jax.experimental.pallas import tpu as pltpu
from jax.experimental.pallas import tpu_sc as plsc

N = 50000
E = 800000
IN_FEATS = 32
H_FEATS = 16
HEADS = 4
HH = HEADS * H_FEATS

NC, NS, L = 2, 16, 16
ROWS_PT = N // NS

BLK = 400
GRID = N // BLK

B1 = 128
EPT1 = E // NS
C1_FULL = EPT1 // B1
R1 = EPT1 - C1_FULL * B1

B2 = 128
EPT2 = E // (NC * NS)
C2_FULL = EPT2 // B2
R2 = EPT2 - C2_FULL * B2
R2P = 48

EPS = 1e-16

DCH = 6256
DLAST = 2 * N - (NS - 1) * DCH

_SC_PARAMS = pltpu.CompilerParams(use_tc_tiling_on_sc=False)


def _copy_flat_slab(s, src_ref, dst_ref):
    @pl.when(s < NS - 1)
    def _():
        pltpu.sync_copy(src_ref.at[pl.ds(s * DCH, DCH)], dst_ref.at[pl.ds(s * DCH, DCH)])

    @pl.when(s == NS - 1)
    def _():
        pltpu.sync_copy(
            src_ref.at[pl.ds((NS - 1) * DCH, DLAST)],
            dst_ref.at[pl.ds((NS - 1) * DCH, DLAST)],
        )


def _lrelu_exp(x):
    return jnp.exp(jnp.maximum(x, 0.2 * x))


def _stage1_body(x_ref, em_ref, *refs):
    x = x_ref[...]
    for i in range(3):
        w_ref, as_ref, ad_ref = refs[3 * i], refs[3 * i + 1], refs[3 * i + 2]
        h_ref, a_ref, d_ref, ia_ref, id_ref = refs[9 + 5 * i : 9 + 5 * i + 5]
        h = jnp.dot(x, w_ref[...], preferred_element_type=jnp.float32, precision=lax.Precision.HIGHEST)
        asrc = jnp.dot(h, as_ref[...], preferred_element_type=jnp.float32, precision=lax.Precision.HIGHEST)
        adst = jnp.dot(h, ad_ref[...], preferred_element_type=jnp.float32, precision=lax.Precision.HIGHEST)
        ex0 = _lrelu_exp(asrc + adst)
        ex64 = jnp.dot(ex0, em_ref[...], preferred_element_type=jnp.float32, precision=lax.Precision.HIGHEST)
        ia = ex64 * h
        h_ref[0] = h[:, :32]
        h_ref[1] = h[:, 32:]
        a_ref[...] = asrc
        d_ref[...] = adst
        ia_ref[0] = ia[:, :32]
        ia_ref[1] = ia[:, 32:]
        id_ref[0] = ex0[:, :2]
        id_ref[1] = ex0[:, 2:]


def _stage1(X, Ws, As, Ads, Em):
    full = lambda shp: pl.BlockSpec(shp, lambda i: (0,) * len(shp))
    in_specs = [pl.BlockSpec((BLK, IN_FEATS), lambda i: (i, 0)), full((HEADS, HH))]
    for _ in range(3):
        in_specs += [full((IN_FEATS, HH)), full((HH, HEADS)), full((HH, HEADS))]
    out_specs, out_shape = [], []
    for _ in range(3):
        for shp, blk in (
            ((NC, N, 32), (NC, BLK, 32)),
            ((N, HEADS), (BLK, HEADS)),
            ((N, HEADS), (BLK, HEADS)),
            ((NC, N, 32), (NC, BLK, 32)),
            ((NC, N, 2), (NC, BLK, 2)),
        ):
            ix = (lambda i: (0, i, 0)) if len(shp) == 3 else (lambda i: (i, 0))
            out_specs.append(pl.BlockSpec(blk, ix))
            out_shape.append(jax.ShapeDtypeStruct(shp, jnp.float32))
    args = [X, Em]
    for i in range(3):
        args += [Ws[i], As[i], Ads[i]]
    return pl.pallas_call(
        _stage1_body,
        grid=(GRID,),
        in_specs=in_specs,
        out_specs=out_specs,
        out_shape=out_shape,
    )(*args)


def _edge_chunk1(src_h, dst_h, asr_h, adt_h, hsp_h, acc_sh, den_sh, bufs, sem,
                 c, eb, nb):
    (sidx, didx, sg0, sg1, dg0, dg1, hidx, dd0, dd1,
     ab0, ab1, db0, db1, ex0, ex1, hbuf, msg) = bufs
    d1 = pltpu.async_copy(src_h.at[pl.ds(eb, nb)], sidx, sem)
    d2_ = pltpu.async_copy(dst_h.at[pl.ds(eb, nb)], didx, sem)
    d1.wait()
    d2_.wait()
    c2 = c * 2
    for g in range(nb // L):
        sl = pl.ds(g * L, L)
        sv = sidx[sl]
        dv = didx[sl]
        s4 = sv * 4 + c2
        d4 = dv * 4 + c2
        sg0[sl] = s4
        sg1[sl] = s4 + 1
        dg0[sl] = d4
        dg1[sl] = d4 + 1
        hidx[sl] = sv + c * N
        d2 = dv * 2
        dd0[sl] = d2
        dd1[sl] = d2 + 1
    gds = [
        pltpu.async_copy(asr_h.at[sg0], ab0, sem),
        pltpu.async_copy(asr_h.at[sg1], ab1, sem),
        pltpu.async_copy(adt_h.at[dg0], db0, sem),
        pltpu.async_copy(adt_h.at[dg1], db1, sem),
        pltpu.async_copy(hsp_h.at[hidx], hbuf, sem),
    ]
    for d in gds:
        d.wait()
    for g in range(nb // L):
        sl = pl.ds(g * L, L)
        ex0[sl] = _lrelu_exp(ab0[sl] + db0[sl])
        ex1[sl] = _lrelu_exp(ab1[sl] + db1[sl])
    def _msg_grp(g, _):
        e0v = ex0[pl.ds(g * L, L)]
        e1v = ex1[pl.ds(g * L, L)]
        for j in range(L):
            r = g * L + j
            msg[r, pl.ds(0, L)] = jnp.full((L,), e0v[j], jnp.float32) * hbuf[r, pl.ds(0, L)]
            msg[r, pl.ds(L, L)] = jnp.full((L,), e1v[j], jnp.float32) * hbuf[r, pl.ds(L, L)]
        return 0

    lax.fori_loop(0, nb // L, _msg_grp, 0)
    sds = [
        pltpu.async_copy(ex0, den_sh.at[dd0], sem, add=True),
        pltpu.async_copy(ex1, den_sh.at[dd1], sem, add=True),
        pltpu.async_copy(msg, acc_sh.at[didx], sem, add=True),
    ]
    for d in sds:
        d.wait()


def _mkbufs1(nb):
    return [pltpu.VMEM((nb,), jnp.int32) for _ in range(9)] + [
        pltpu.VMEM((nb,), jnp.float32) for _ in range(6)
    ] + [pltpu.VMEM((nb, 32), jnp.float32) for _ in range(2)]


def _gat1_edges(src, dst, asr, adt, hsp, iacc, iden):
    mesh = plsc.VectorSubcoreMesh(core_axis_name="c", subcore_axis_name="s")

    @functools.partial(
        pl.kernel,
        out_type=(
            jax.ShapeDtypeStruct((NC, N, 32), jnp.float32),
            jax.ShapeDtypeStruct((NC, 2 * N), jnp.float32),
        ),
        mesh=mesh,
        scratch_types=[
            pltpu.VMEM_SHARED((N, 32), jnp.float32),
            pltpu.VMEM_SHARED((2 * N,), jnp.float32),
        ]
        + _mkbufs1(B1)
        + _mkbufs1(R1)
        + [pltpu.SemaphoreType.DMA],
        compiler_params=_SC_PARAMS,
    )
    def k(src_h, dst_h, asr_h, adt_h, hsp_h, iacc_h, iden_h, acc_out, den_out,
          acc_sh, den_sh, *scr):
        bufs = scr[:17]
        bufr = scr[17:34]
        sem = scr[34]
        c = lax.axis_index("c")
        s = lax.axis_index("s")
        r0 = s * ROWS_PT
        pltpu.sync_copy(iacc_h.at[c, pl.ds(r0, ROWS_PT)], acc_sh.at[pl.ds(r0, ROWS_PT)])
        _copy_flat_slab(s, iden_h.at[c], den_sh)
        plsc.subcore_barrier()
        base = s * EPT1

        def body(j, _):
            _edge_chunk1(src_h, dst_h, asr_h, adt_h, hsp_h, acc_sh, den_sh,
                         bufs, sem, c, base + j * B1, B1)
            return 0

        lax.fori_loop(0, C1_FULL, body, 0)
        _edge_chunk1(src_h, dst_h, asr_h, adt_h, hsp_h, acc_sh, den_sh,
                     bufr, sem, c, base + C1_FULL * B1, R1)
        plsc.subcore_barrier()
        pltpu.sync_copy(acc_sh.at[pl.ds(r0, ROWS_PT)], acc_out.at[c, pl.ds(r0, ROWS_PT)])
        _copy_flat_slab(s, den_sh, den_out.at[c])

    return k(src, dst, asr, adt, hsp, iacc, iden)


def _stage2_body(e2_ref, *refs):
    xs = []
    for i in range(3):
        acc_ref, den_ref, b1_ref = refs[5 * i], refs[5 * i + 1], refs[5 * i + 2]
        parts = []
        for cc in range(NC):
            den32 = jnp.dot(den_ref[cc], e2_ref[...], preferred_element_type=jnp.float32, precision=lax.Precision.HIGHEST)
            parts.append(acc_ref[cc] / (den32 + EPS) + b1_ref[0, 32 * cc : 32 * cc + 32])
        xs.append(jax.nn.relu(jnp.concatenate(parts, axis=-1)))
    xc = jnp.concatenate(xs, axis=-1)
    for i in range(3):
        w2_ref, sc_ref = refs[5 * i + 3], refs[5 * i + 4]
        s2_ref, h2o_ref, d2_ref, it_ref = refs[15 + 4 * i : 15 + 4 * i + 4]
        h2 = jnp.dot(xc, w2_ref[...], preferred_element_type=jnp.float32, precision=lax.Precision.HIGHEST)
        a2s = sc_ref[0, 0] * h2
        a2d = sc_ref[0, 1] * h2
        e0 = _lrelu_exp(a2s + a2d)
        s2_ref[...] = a2s
        h2o_ref[...] = h2
        d2_ref[...] = a2d
        it_ref[0] = jnp.concatenate([e0, e0 * h2], axis=-1)
        it_ref[1] = jnp.zeros((h2.shape[0], 2), jnp.float32)


def _stage2(accs, dens, b1s, w2s, scs, E2):
    full = lambda shp: pl.BlockSpec(shp, lambda i: (0,) * len(shp))
    in_specs = [full((2, 32))]
    args = [E2]
    for i in range(3):
        in_specs += [
            pl.BlockSpec((NC, BLK, 32), lambda i: (0, i, 0)),
            pl.BlockSpec((NC, BLK, 2), lambda i: (0, i, 0)),
            full((1, HH)),
            full((3 * HH, 1)),
            full((1, 2)),
        ]
        args += [accs[i], dens[i], b1s[i], w2s[i], scs[i]]
    out_specs, out_shape = [], []
    for _ in range(3):
        out_specs += [
            pl.BlockSpec((BLK, 1), lambda i: (i, 0)),
            pl.BlockSpec((BLK, 1), lambda i: (i, 0)),
            pl.BlockSpec((BLK, 1), lambda i: (i, 0)),
            pl.BlockSpec((NC, BLK, 2), lambda i: (0, i, 0)),
        ]
        out_shape += [
            jax.ShapeDtypeStruct((N, 1), jnp.float32),
            jax.ShapeDtypeStruct((N, 1), jnp.float32),
            jax.ShapeDtypeStruct((N, 1), jnp.float32),
            jax.ShapeDtypeStruct((NC, N, 2), jnp.float32),
        ]
    return pl.pallas_call(
        _stage2_body,
        grid=(GRID,),
        in_specs=in_specs,
        out_specs=out_specs,
        out_shape=out_shape,
    )(*args)


def _edge_chunk2(src_h, dst_h, s2a_h, s2h_h, d2a_h, t2_sh, bufs, sem, eb, nb,
                 valid=None):
    sidx, didx, dd0, dd1, ab, hb, db, exb, nmb = bufs
    nv = nb if valid is None else valid
    l1 = pltpu.async_copy(src_h.at[pl.ds(eb, nv)], sidx.at[pl.ds(0, nv)], sem)
    l2 = pltpu.async_copy(dst_h.at[pl.ds(eb, nv)], didx.at[pl.ds(0, nv)], sem)
    l1.wait()
    l2.wait()
    iot = lax.iota(jnp.int32, L)
    for g in range(nb // L):
        sl = pl.ds(g * L, L)
        if valid is not None:
            m = (g * L + iot) < valid
            sidx[sl] = jnp.where(m, sidx[sl], 0)
            didx[sl] = jnp.where(m, didx[sl], 0)
        d2 = didx[sl] * 2
        dd0[sl] = d2
        dd1[sl] = d2 + 1
    gds = [
        pltpu.async_copy(s2a_h.at[sidx], ab, sem),
        pltpu.async_copy(s2h_h.at[sidx], hb, sem),
        pltpu.async_copy(d2a_h.at[didx], db, sem),
    ]
    for d in gds:
        d.wait()
    for g in range(nb // L):
        sl = pl.ds(g * L, L)
        ex = _lrelu_exp(ab[sl] + db[sl])
        if valid is not None:
            ex = jnp.where((g * L + iot) < valid, ex, 0.0)
        exb[sl] = ex
        nmb[sl] = ex * hb[sl]
    sds = [
        pltpu.async_copy(exb, t2_sh.at[dd0], sem, add=True),
        pltpu.async_copy(nmb, t2_sh.at[dd1], sem, add=True),
    ]
    for d in sds:
        d.wait()


def _mkbufs2(nb):
    return [pltpu.VMEM((nb,), jnp.int32) for _ in range(4)] + [
        pltpu.VMEM((nb,), jnp.float32) for _ in range(5)
    ]


def _gat2_edges(srcs, dsts, s2as, s2hs, d2as, its):
    mesh = plsc.VectorSubcoreMesh(core_axis_name="c", subcore_axis_name="s")

    @functools.partial(
        pl.kernel,
        out_type=tuple(
            jax.ShapeDtypeStruct((NC, 2 * N), jnp.float32) for _ in range(3)
        ),
        mesh=mesh,
        scratch_types=[pltpu.VMEM_SHARED((2 * N,), jnp.float32) for _ in range(3)]
        + _mkbufs2(B2)
        + _mkbufs2(R2P)
        + [pltpu.SemaphoreType.DMA],
        compiler_params=_SC_PARAMS,
    )
    def k(s1, d1, s2, d2, s3, d3, a1, h1, q1, a2, h2, q2, a3, h3, q3,
          i1, i2, i3, o1, o2, o3, t1_sh, t2_sh, t3_sh, *scr):
        bufs = scr[:9]
        bufr = scr[9:18]
        sem = scr[18]
        c = lax.axis_index("c")
        s = lax.axis_index("s")
        r0 = s * ROWS_PT
        srcl, dstl = (s1, s2, s3), (d1, d2, d3)
        tabl = ((a1, h1, q1), (a2, h2, q2), (a3, h3, q3))
        itl = (i1, i2, i3)
        outl = (o1, o2, o3)
        shl = (t1_sh, t2_sh, t3_sh)
        for a in range(3):
            _copy_flat_slab(s, itl[a].at[c], shl[a])
        plsc.subcore_barrier()
        base = c * (E // NC) + s * EPT2
        for a in range(3):
            def body(j, _, a=a):
                _edge_chunk2(srcl[a], dstl[a], *tabl[a], shl[a], bufs, sem,
                             base + j * B2, B2)
                return 0

            lax.fori_loop(0, C2_FULL, body, 0)
            _edge_chunk2(srcl[a], dstl[a], *tabl[a], shl[a], bufr, sem,
                         base + C2_FULL * B2, R2P, valid=R2)
        plsc.subcore_barrier()
        for a in range(3):
            _copy_flat_slab(s, shl[a], outl[a].at[c])

    return k(srcs[0], dsts[0], srcs[1], dsts[1], srcs[2], dsts[2],
             s2as[0], s2hs[0], d2as[0], s2as[1], s2hs[1], d2as[1],
             s2as[2], s2hs[2], d2as[2], its[0], its[1], its[2])


def _final_body(t1_ref, t2_ref, t3_ref, cv_ref, out_ref):
    cv = cv_ref[...]
    acc = None
    for i, t_ref in enumerate((t1_ref, t2_ref, t3_ref)):
        t = t_ref[0] + t_ref[1]
        y = t[:, 1:2] / (t[:, 0:1] + EPS) + cv[0, i]
        term = y * cv[0, 3 + i]
        acc = term if acc is None else acc + term
    out_ref[...] = acc + cv[0, 6]


def _final(t2outs, cvec):
    blk3 = pl.BlockSpec((NC, BLK, 2), lambda i: (0, i, 0))
    return pl.pallas_call(
        _final_body,
        grid=(GRID,),
        in_specs=[blk3, blk3, blk3, pl.BlockSpec((1, 8), lambda i: (0, 0))],
        out_specs=pl.BlockSpec((BLK, 1), lambda i: (i, 0)),
        out_shape=jax.ShapeDtypeStruct((N, 1), jnp.float32),
    )(*t2outs, cvec)


def kernel(X, A1, A2, A3, edge_feature, params):
    p = params
    As, Ads, Ws, b1s, w2s, scs = [], [], [], [], [], []
    for i in (1, 2, 3):
        a_s = p[f"as1_{i}"].reshape(HEADS, H_FEATS)
        a_d = p[f"ad1_{i}"].reshape(HEADS, H_FEATS)
        zer = jnp.zeros((HEADS, HH), jnp.float32)
        rows = jnp.arange(HEADS)[:, None]
        cols = rows * H_FEATS + jnp.arange(H_FEATS)[None, :]
        As.append(zer.at[rows, cols].set(a_s).T)
        Ads.append(zer.at[rows, cols].set(a_d).T)
        Ws.append(p[f"W1_{i}"])
        b1s.append(p[f"b1_{i}"].reshape(1, HH))
        w2s.append(p[f"W2_{i}"])
        scs.append(jnp.stack([p[f"as2_{i}"].reshape(()), p[f"ad2_{i}"].reshape(())]).reshape(1, 2))
    Em = jnp.zeros((HEADS, HH), jnp.float32).at[
        jnp.arange(HEADS)[:, None],
        jnp.arange(HEADS)[:, None] * H_FEATS + jnp.arange(H_FEATS)[None, :],
    ].set(1.0)
    E2 = jnp.zeros((2, 32), jnp.float32).at[
        jnp.arange(2)[:, None],
        jnp.arange(2)[:, None] * H_FEATS + jnp.arange(H_FEATS)[None, :],
    ].set(1.0)

    s1 = _stage1(X, Ws, As, Ads, Em)
    accs, dens = [], []
    srcs, dsts = [], []
    for i, A in enumerate((A1, A2, A3)):
        H, Aq, D, IA, ID = s1[5 * i : 5 * i + 5]
        src = A[0].astype(jnp.int32)
        dst = A[1].astype(jnp.int32)
        srcs.append(src)
        dsts.append(dst)
        acc, den = _gat1_edges(
            src, dst,
            Aq.reshape(N * HEADS), D.reshape(N * HEADS), H.reshape(NC * N, 32),
            IA, ID.reshape(NC, 2 * N),
        )
        accs.append(acc)
        dens.append(den.reshape(NC, N, 2))

    s2 = _stage2(accs, dens, b1s, w2s, scs, E2)
    s2as = [s2[4 * i].reshape(N) for i in range(3)]
    s2hs = [s2[4 * i + 1].reshape(N) for i in range(3)]
    d2as = [s2[4 * i + 2].reshape(N) for i in range(3)]
    its = [s2[4 * i + 3].reshape(NC, 2 * N) for i in range(3)]

    t2outs = _gat2_edges(srcs, dsts, s2as, s2hs, d2as, its)

    cvec = jnp.concatenate(
        [
            jnp.stack([p["b2_1"][0], p["b2_2"][0], p["b2_3"][0]]),
            p["Wln"][:, 0],
            p["bln"],
            jnp.zeros((1,), jnp.float32),
        ]
    ).reshape(1, 8)
    return _final([t.reshape(NC, N, 2) for t in t2outs], cvec)


# speedup vs baseline: 94.3977x; 1.5384x over previous
import functools

import jax
import jax.numpy as jnp
from jax import lax
from jax.experimental import pallas as pl
from jax.experimental.pallas import tpu as pltpu
from jax.experimental.pallas import tpu_sc as plsc

N = 50000
E = 800000
IN_FEATS = 32
H_FEATS = 16
HEADS = 4
HH = HEADS * H_FEATS

NC, NS, L = 2, 16, 16
ROWS_PT = N // NS

BLK = 400
GRID = N // BLK

B1 = 128
EPT1 = E // NS
C1_FULL = EPT1 // B1
R1 = EPT1 - C1_FULL * B1

B2 = 128
EPT2 = E // (NC * NS)
C2_FULL = EPT2 // B2
R2 = EPT2 - C2_FULL * B2
R2P = 48

EPS = 1e-16

DCH = 6256
DLAST = 2 * N - (NS - 1) * DCH

_SC_PARAMS = pltpu.CompilerParams(use_tc_tiling_on_sc=False)


def _copy_flat_slab(s, src_ref, dst_ref):
    @pl.when(s < NS - 1)
    def _():
        pltpu.sync_copy(src_ref.at[pl.ds(s * DCH, DCH)], dst_ref.at[pl.ds(s * DCH, DCH)])

    @pl.when(s == NS - 1)
    def _():
        pltpu.sync_copy(
            src_ref.at[pl.ds((NS - 1) * DCH, DLAST)],
            dst_ref.at[pl.ds((NS - 1) * DCH, DLAST)],
        )


def _lrelu_exp(x):
    return jnp.exp(jnp.maximum(x, 0.2 * x))


def _stage1_body(x_ref, em_ref, *refs):
    x = x_ref[...]
    for i in range(3):
        w_ref, as_ref, ad_ref = refs[3 * i], refs[3 * i + 1], refs[3 * i + 2]
        h_ref, a_ref, d_ref, ia_ref, id_ref = refs[9 + 5 * i : 9 + 5 * i + 5]
        h = jnp.dot(x, w_ref[...], preferred_element_type=jnp.float32, precision=lax.Precision.HIGHEST)
        asrc = jnp.dot(h, as_ref[...], preferred_element_type=jnp.float32, precision=lax.Precision.HIGHEST)
        adst = jnp.dot(h, ad_ref[...], preferred_element_type=jnp.float32, precision=lax.Precision.HIGHEST)
        ex0 = _lrelu_exp(asrc + adst)
        ex64 = jnp.dot(ex0, em_ref[...], preferred_element_type=jnp.float32, precision=lax.Precision.HIGHEST)
        ia = ex64 * h
        h_ref[0] = h[:, :32]
        h_ref[1] = h[:, 32:]
        a_ref[...] = asrc
        d_ref[...] = adst
        ia_ref[0] = ia[:, :32]
        ia_ref[1] = ia[:, 32:]
        id_ref[0] = ex0[:, :2]
        id_ref[1] = ex0[:, 2:]


def _stage1(X, Ws, As, Ads, Em):
    full = lambda shp: pl.BlockSpec(shp, lambda i: (0,) * len(shp))
    in_specs = [pl.BlockSpec((BLK, IN_FEATS), lambda i: (i, 0)), full((HEADS, HH))]
    for _ in range(3):
        in_specs += [full((IN_FEATS, HH)), full((HH, HEADS)), full((HH, HEADS))]
    out_specs, out_shape = [], []
    for _ in range(3):
        for shp, blk in (
            ((NC, N, 32), (NC, BLK, 32)),
            ((N, HEADS), (BLK, HEADS)),
            ((N, HEADS), (BLK, HEADS)),
            ((NC, N, 32), (NC, BLK, 32)),
            ((NC, N, 2), (NC, BLK, 2)),
        ):
            ix = (lambda i: (0, i, 0)) if len(shp) == 3 else (lambda i: (i, 0))
            out_specs.append(pl.BlockSpec(blk, ix))
            out_shape.append(jax.ShapeDtypeStruct(shp, jnp.float32))
    args = [X, Em]
    for i in range(3):
        args += [Ws[i], As[i], Ads[i]]
    return pl.pallas_call(
        _stage1_body,
        grid=(GRID,),
        in_specs=in_specs,
        out_specs=out_specs,
        out_shape=out_shape,
    )(*args)


def _prep1(src_h, dst_h, asr_h, adt_h, hsp_h, bufs, gsem, c, eb, nb):
    (sidx, didx, sg0, sg1, dg0, dg1, hidx, dd0, dd1, didxm,
     ab0, ab1, db0, db1, ex0, ex1, hbuf, msg) = bufs
    u = pltpu.async_copy(src_h.at[pl.ds(eb, nb)], sidx, gsem)
    v = pltpu.async_copy(dst_h.at[pl.ds(eb, nb)], didx, gsem)
    u.wait()
    v.wait()
    c2 = c * 2
    for g in range(nb // L):
        sl = pl.ds(g * L, L)
        sv = sidx[sl]
        dv = didx[sl]
        s4 = sv * 4 + c2
        d4 = dv * 4 + c2
        sg0[sl] = s4
        sg1[sl] = s4 + 1
        dg0[sl] = d4
        dg1[sl] = d4 + 1
        hidx[sl] = sv + c * N
    pltpu.async_copy(asr_h.at[sg0], ab0, gsem)
    pltpu.async_copy(asr_h.at[sg1], ab1, gsem)
    pltpu.async_copy(adt_h.at[dg0], db0, gsem)
    pltpu.async_copy(adt_h.at[dg1], db1, gsem)
    pltpu.async_copy(hsp_h.at[hidx], hbuf, gsem)


def _gdrain1(asr_h, adt_h, hsp_h, bufs, gsem):
    (sidx, didx, sg0, sg1, dg0, dg1, hidx, dd0, dd1, didxm,
     ab0, ab1, db0, db1, ex0, ex1, hbuf, msg) = bufs
    pltpu.make_async_copy(asr_h.at[sg0], ab0, gsem).wait()
    pltpu.make_async_copy(asr_h.at[sg1], ab1, gsem).wait()
    pltpu.make_async_copy(adt_h.at[dg0], db0, gsem).wait()
    pltpu.make_async_copy(adt_h.at[dg1], db1, gsem).wait()
    pltpu.make_async_copy(hsp_h.at[hidx], hbuf, gsem).wait()


def _sdrain1(acc_sh, den_sh, bufs, ssem):
    (sidx, didx, sg0, sg1, dg0, dg1, hidx, dd0, dd1, didxm,
     ab0, ab1, db0, db1, ex0, ex1, hbuf, msg) = bufs
    pltpu.make_async_copy(ex0, den_sh.at[dd0], ssem).wait()
    pltpu.make_async_copy(ex1, den_sh.at[dd1], ssem).wait()
    pltpu.make_async_copy(msg, acc_sh.at[didxm], ssem).wait()


def _proc1(acc_sh, den_sh, bufs, ssem, nb):
    (sidx, didx, sg0, sg1, dg0, dg1, hidx, dd0, dd1, didxm,
     ab0, ab1, db0, db1, ex0, ex1, hbuf, msg) = bufs
    for g in range(nb // L):
        sl = pl.ds(g * L, L)
        dv = didx[sl]
        d2 = dv * 2
        dd0[sl] = d2
        dd1[sl] = d2 + 1
        didxm[sl] = dv
        ex0[sl] = _lrelu_exp(ab0[sl] + db0[sl])
        ex1[sl] = _lrelu_exp(ab1[sl] + db1[sl])

    def _msg_grp(g, _):
        e0v = ex0[pl.ds(g * L, L)]
        e1v = ex1[pl.ds(g * L, L)]
        for j in range(L):
            r = g * L + j
            msg[r, pl.ds(0, L)] = jnp.full((L,), e0v[j], jnp.float32) * hbuf[r, pl.ds(0, L)]
            msg[r, pl.ds(L, L)] = jnp.full((L,), e1v[j], jnp.float32) * hbuf[r, pl.ds(L, L)]
        return 0

    lax.fori_loop(0, nb // L, _msg_grp, 0)
    pltpu.async_copy(ex0, den_sh.at[dd0], ssem, add=True)
    pltpu.async_copy(ex1, den_sh.at[dd1], ssem, add=True)
    pltpu.async_copy(msg, acc_sh.at[didxm], ssem, add=True)


def _edge_chunk1(src_h, dst_h, asr_h, adt_h, hsp_h, acc_sh, den_sh, bufs, sem,
                 c, eb, nb, valid):
    (sidx, didx, sg0, sg1, dg0, dg1, hidx, dd0, dd1, didxm,
     ab0, ab1, db0, db1, ex0, ex1, hbuf, msg) = bufs
    d1 = pltpu.async_copy(src_h.at[pl.ds(eb, valid)], sidx.at[pl.ds(0, valid)], sem)
    d2_ = pltpu.async_copy(dst_h.at[pl.ds(eb, valid)], didx.at[pl.ds(0, valid)], sem)
    d1.wait()
    d2_.wait()
    c2 = c * 2
    iot = lax.iota(jnp.int32, L)
    for g in range(nb // L):
        sl = pl.ds(g * L, L)
        m = (g * L + iot) < valid
        sv = jnp.where(m, sidx[sl], 0)
        dv = jnp.where(m, didx[sl], 0)
        sidx[sl] = sv
        didx[sl] = dv
        s4 = sv * 4 + c2
        d4 = dv * 4 + c2
        sg0[sl] = s4
        sg1[sl] = s4 + 1
        dg0[sl] = d4
        dg1[sl] = d4 + 1
        hidx[sl] = sv + c * N
        d2 = dv * 2
        dd0[sl] = d2
        dd1[sl] = d2 + 1
    gds = [
        pltpu.async_copy(asr_h.at[sg0], ab0, sem),
        pltpu.async_copy(asr_h.at[sg1], ab1, sem),
        pltpu.async_copy(adt_h.at[dg0], db0, sem),
        pltpu.async_copy(adt_h.at[dg1], db1, sem),
        pltpu.async_copy(hsp_h.at[hidx], hbuf, sem),
    ]
    for d in gds:
        d.wait()
    for g in range(nb // L):
        sl = pl.ds(g * L, L)
        m = (g * L + iot) < valid
        ex0[sl] = jnp.where(m, _lrelu_exp(ab0[sl] + db0[sl]), 0.0)
        ex1[sl] = jnp.where(m, _lrelu_exp(ab1[sl] + db1[sl]), 0.0)

    def _msg_grp(g, _):
        e0v = ex0[pl.ds(g * L, L)]
        e1v = ex1[pl.ds(g * L, L)]
        for j in range(L):
            r = g * L + j
            msg[r, pl.ds(0, L)] = jnp.full((L,), e0v[j], jnp.float32) * hbuf[r, pl.ds(0, L)]
            msg[r, pl.ds(L, L)] = jnp.full((L,), e1v[j], jnp.float32) * hbuf[r, pl.ds(L, L)]
        return 0

    lax.fori_loop(0, nb // L, _msg_grp, 0)
    sds = [
        pltpu.async_copy(ex0, den_sh.at[dd0], sem, add=True),
        pltpu.async_copy(ex1, den_sh.at[dd1], sem, add=True),
        pltpu.async_copy(msg, acc_sh.at[didx], sem, add=True),
    ]
    for d in sds:
        d.wait()


def _mkbufs1(nb):
    return [pltpu.VMEM((nb,), jnp.int32) for _ in range(10)] + [
        pltpu.VMEM((nb,), jnp.float32) for _ in range(6)
    ] + [pltpu.VMEM((nb, 32), jnp.float32) for _ in range(2)]


def _gat1_edges(src, dst, asr, adt, hsp, iacc, iden):
    mesh = plsc.VectorSubcoreMesh(core_axis_name="c", subcore_axis_name="s")

    @functools.partial(
        pl.kernel,
        out_type=(
            jax.ShapeDtypeStruct((NC, N, 32), jnp.float32),
            jax.ShapeDtypeStruct((NC, 2 * N), jnp.float32),
        ),
        mesh=mesh,
        scratch_types=[
            pltpu.VMEM_SHARED((N, 32), jnp.float32),
            pltpu.VMEM_SHARED((2 * N,), jnp.float32),
        ]
        + _mkbufs1(B1)
        + _mkbufs1(B1)
        + [pltpu.SemaphoreType.DMA for _ in range(5)],
        compiler_params=_SC_PARAMS,
    )
    def k(src_h, dst_h, asr_h, adt_h, hsp_h, iacc_h, iden_h, acc_out, den_out,
          acc_sh, den_sh, *scr):
        bufa = scr[:18]
        bufb = scr[18:36]
        g0, g1, s0, s1, sem = scr[36:41]
        c = lax.axis_index("c")
        s = lax.axis_index("s")
        r0 = s * ROWS_PT
        pltpu.sync_copy(iacc_h.at[c, pl.ds(r0, ROWS_PT)], acc_sh.at[pl.ds(r0, ROWS_PT)])
        _copy_flat_slab(s, iden_h.at[c], den_sh)
        plsc.subcore_barrier()
        base = s * EPT1

        _prep1(src_h, dst_h, asr_h, adt_h, hsp_h, bufa, g0, c, base, B1)
        _prep1(src_h, dst_h, asr_h, adt_h, hsp_h, bufb, g1, c, base + B1, B1)

        def body(k2, _):
            for p, (bufs, gsem, ssem) in enumerate(((bufa, g0, s0), (bufb, g1, s1))):
                j = 2 * k2 + p
                _gdrain1(asr_h, adt_h, hsp_h, bufs, gsem)

                @pl.when(k2 > 0)
                def _():
                    _sdrain1(acc_sh, den_sh, bufs, ssem)

                _proc1(acc_sh, den_sh, bufs, ssem, B1)

                @pl.when(j + 2 < C1_FULL)
                def _():
                    _prep1(src_h, dst_h, asr_h, adt_h, hsp_h, bufs, gsem, c,
                           base + (j + 2) * B1, B1)

            return 0

        lax.fori_loop(0, C1_FULL // 2, body, 0)
        _sdrain1(acc_sh, den_sh, bufa, s0)
        _sdrain1(acc_sh, den_sh, bufb, s1)
        _edge_chunk1(src_h, dst_h, asr_h, adt_h, hsp_h, acc_sh, den_sh,
                     bufa, sem, c, base + C1_FULL * B1, B1, R1)
        plsc.subcore_barrier()
        pltpu.sync_copy(acc_sh.at[pl.ds(r0, ROWS_PT)], acc_out.at[c, pl.ds(r0, ROWS_PT)])
        _copy_flat_slab(s, den_sh, den_out.at[c])

    return k(src, dst, asr, adt, hsp, iacc, iden)


def _stage2_body(e2_ref, *refs):
    xs = []
    for i in range(3):
        acc_ref, den_ref, b1_ref = refs[5 * i], refs[5 * i + 1], refs[5 * i + 2]
        parts = []
        for cc in range(NC):
            den32 = jnp.dot(den_ref[cc], e2_ref[...], preferred_element_type=jnp.float32, precision=lax.Precision.HIGHEST)
            parts.append(acc_ref[cc] / (den32 + EPS) + b1_ref[0, 32 * cc : 32 * cc + 32])
        xs.append(jax.nn.relu(jnp.concatenate(parts, axis=-1)))
    xc = jnp.concatenate(xs, axis=-1)
    for i in range(3):
        w2_ref, sc_ref = refs[5 * i + 3], refs[5 * i + 4]
        s2_ref, h2o_ref, d2_ref, it_ref = refs[15 + 4 * i : 15 + 4 * i + 4]
        h2 = jnp.dot(xc, w2_ref[...], preferred_element_type=jnp.float32, precision=lax.Precision.HIGHEST)
        a2s = sc_ref[0, 0] * h2
        a2d = sc_ref[0, 1] * h2
        e0 = _lrelu_exp(a2s + a2d)
        s2_ref[...] = a2s
        h2o_ref[...] = h2
        d2_ref[...] = a2d
        it_ref[0] = jnp.concatenate([e0, e0 * h2], axis=-1)
        it_ref[1] = jnp.zeros((h2.shape[0], 2), jnp.float32)


def _stage2(accs, dens, b1s, w2s, scs, E2):
    full = lambda shp: pl.BlockSpec(shp, lambda i: (0,) * len(shp))
    in_specs = [full((2, 32))]
    args = [E2]
    for i in range(3):
        in_specs += [
            pl.BlockSpec((NC, BLK, 32), lambda i: (0, i, 0)),
            pl.BlockSpec((NC, BLK, 2), lambda i: (0, i, 0)),
            full((1, HH)),
            full((3 * HH, 1)),
            full((1, 2)),
        ]
        args += [accs[i], dens[i], b1s[i], w2s[i], scs[i]]
    out_specs, out_shape = [], []
    for _ in range(3):
        out_specs += [
            pl.BlockSpec((BLK, 1), lambda i: (i, 0)),
            pl.BlockSpec((BLK, 1), lambda i: (i, 0)),
            pl.BlockSpec((BLK, 1), lambda i: (i, 0)),
            pl.BlockSpec((NC, BLK, 2), lambda i: (0, i, 0)),
        ]
        out_shape += [
            jax.ShapeDtypeStruct((N, 1), jnp.float32),
            jax.ShapeDtypeStruct((N, 1), jnp.float32),
            jax.ShapeDtypeStruct((N, 1), jnp.float32),
            jax.ShapeDtypeStruct((NC, N, 2), jnp.float32),
        ]
    return pl.pallas_call(
        _stage2_body,
        grid=(GRID,),
        in_specs=in_specs,
        out_specs=out_specs,
        out_shape=out_shape,
    )(*args)


def _prep2(src_h, dst_h, s2a_h, s2h_h, d2a_h, bufs, gsem, eb, nb):
    sidx, didx, dd0, dd1, ab, hb, db, exb, nmb = bufs
    u = pltpu.async_copy(src_h.at[pl.ds(eb, nb)], sidx, gsem)
    v = pltpu.async_copy(dst_h.at[pl.ds(eb, nb)], didx, gsem)
    u.wait()
    v.wait()
    pltpu.async_copy(s2a_h.at[sidx], ab, gsem)
    pltpu.async_copy(s2h_h.at[sidx], hb, gsem)
    pltpu.async_copy(d2a_h.at[didx], db, gsem)


def _gdrain2(s2a_h, s2h_h, d2a_h, bufs, gsem):
    sidx, didx, dd0, dd1, ab, hb, db, exb, nmb = bufs
    pltpu.make_async_copy(s2a_h.at[sidx], ab, gsem).wait()
    pltpu.make_async_copy(s2h_h.at[sidx], hb, gsem).wait()
    pltpu.make_async_copy(d2a_h.at[didx], db, gsem).wait()


def _sdrain2(t2_sh, bufs, ssem):
    sidx, didx, dd0, dd1, ab, hb, db, exb, nmb = bufs
    pltpu.make_async_copy(exb, t2_sh.at[dd0], ssem).wait()
    pltpu.make_async_copy(nmb, t2_sh.at[dd1], ssem).wait()


def _proc2(t2_sh, bufs, ssem, nb):
    sidx, didx, dd0, dd1, ab, hb, db, exb, nmb = bufs
    for g in range(nb // L):
        sl = pl.ds(g * L, L)
        d2 = didx[sl] * 2
        dd0[sl] = d2
        dd1[sl] = d2 + 1
        ex = _lrelu_exp(ab[sl] + db[sl])
        exb[sl] = ex
        nmb[sl] = ex * hb[sl]
    pltpu.async_copy(exb, t2_sh.at[dd0], ssem, add=True)
    pltpu.async_copy(nmb, t2_sh.at[dd1], ssem, add=True)


def _edge_chunk2(src_h, dst_h, s2a_h, s2h_h, d2a_h, t2_sh, bufs, sem, eb, nb,
                 valid=None):
    sidx, didx, dd0, dd1, ab, hb, db, exb, nmb = bufs
    nv = nb if valid is None else valid
    l1 = pltpu.async_copy(src_h.at[pl.ds(eb, nv)], sidx.at[pl.ds(0, nv)], sem)
    l2 = pltpu.async_copy(dst_h.at[pl.ds(eb, nv)], didx.at[pl.ds(0, nv)], sem)
    l1.wait()
    l2.wait()
    iot = lax.iota(jnp.int32, L)
    for g in range(nb // L):
        sl = pl.ds(g * L, L)
        if valid is not None:
            m = (g * L + iot) < valid
            sidx[sl] = jnp.where(m, sidx[sl], 0)
            didx[sl] = jnp.where(m, didx[sl], 0)
        d2 = didx[sl] * 2
        dd0[sl] = d2
        dd1[sl] = d2 + 1
    gds = [
        pltpu.async_copy(s2a_h.at[sidx], ab, sem),
        pltpu.async_copy(s2h_h.at[sidx], hb, sem),
        pltpu.async_copy(d2a_h.at[didx], db, sem),
    ]
    for d in gds:
        d.wait()
    for g in range(nb // L):
        sl = pl.ds(g * L, L)
        ex = _lrelu_exp(ab[sl] + db[sl])
        if valid is not None:
            ex = jnp.where((g * L + iot) < valid, ex, 0.0)
        exb[sl] = ex
        nmb[sl] = ex * hb[sl]
    sds = [
        pltpu.async_copy(exb, t2_sh.at[dd0], sem, add=True),
        pltpu.async_copy(nmb, t2_sh.at[dd1], sem, add=True),
    ]
    for d in sds:
        d.wait()


def _mkbufs2(nb):
    return [pltpu.VMEM((nb,), jnp.int32) for _ in range(4)] + [
        pltpu.VMEM((nb,), jnp.float32) for _ in range(5)
    ]


def _gat2_edges(srcs, dsts, s2as, s2hs, d2as, its):
    mesh = plsc.VectorSubcoreMesh(core_axis_name="c", subcore_axis_name="s")

    @functools.partial(
        pl.kernel,
        out_type=tuple(
            jax.ShapeDtypeStruct((NC, 2 * N), jnp.float32) for _ in range(3)
        ),
        mesh=mesh,
        scratch_types=[pltpu.VMEM_SHARED((2 * N,), jnp.float32) for _ in range(3)]
        + _mkbufs2(B2)
        + _mkbufs2(B2)
        + _mkbufs2(R2P)
        + [pltpu.SemaphoreType.DMA for _ in range(5)],
        compiler_params=_SC_PARAMS,
    )
    def k(s1, d1, s2, d2, s3, d3, a1, h1, q1, a2, h2, q2, a3, h3, q3,
          i1, i2, i3, o1, o2, o3, t1_sh, t2_sh, t3_sh, *scr):
        bufa = scr[:9]
        bufb = scr[9:18]
        bufr = scr[18:27]
        g0, g1, s0sem, s1sem, sem = scr[27:32]
        c = lax.axis_index("c")
        s = lax.axis_index("s")
        r0 = s * ROWS_PT
        srcl, dstl = (s1, s2, s3), (d1, d2, d3)
        tabl = ((a1, h1, q1), (a2, h2, q2), (a3, h3, q3))
        itl = (i1, i2, i3)
        outl = (o1, o2, o3)
        shl = (t1_sh, t2_sh, t3_sh)
        for a in range(3):
            _copy_flat_slab(s, itl[a].at[c], shl[a])
        plsc.subcore_barrier()
        base = c * (E // NC) + s * EPT2
        n_pipe = C2_FULL - 1
        for a in range(3):
            sa, da = srcl[a], dstl[a]
            ta, th, tq = tabl[a]
            sh = shl[a]
            _prep2(sa, da, ta, th, tq, bufa, g0, base, B2)
            _prep2(sa, da, ta, th, tq, bufb, g1, base + B2, B2)

            def body(k2, _, sa=sa, da=da, ta=ta, th=th, tq=tq, sh=sh):
                for p, (bufs, gsem, ssem) in enumerate(
                    ((bufa, g0, s0sem), (bufb, g1, s1sem))
                ):
                    j = 2 * k2 + p
                    _gdrain2(ta, th, tq, bufs, gsem)

                    @pl.when(k2 > 0)
                    def _():
                        _sdrain2(sh, bufs, ssem)

                    _proc2(sh, bufs, ssem, B2)

                    @pl.when(j + 2 < n_pipe)
                    def _():
                        _prep2(sa, da, ta, th, tq, bufs, gsem,
                               base + (j + 2) * B2, B2)

                return 0

            lax.fori_loop(0, n_pipe // 2, body, 0)
            _sdrain2(sh, bufa, s0sem)
            _sdrain2(sh, bufb, s1sem)
            _edge_chunk2(sa, da, ta, th, tq, sh, bufa, sem,
                         base + n_pipe * B2, B2)
            _edge_chunk2(sa, da, ta, th, tq, sh, bufr, sem,
                         base + C2_FULL * B2, R2P, valid=R2)
        plsc.subcore_barrier()
        for a in range(3):
            _copy_flat_slab(s, shl[a], outl[a].at[c])

    return k(srcs[0], dsts[0], srcs[1], dsts[1], srcs[2], dsts[2],
             s2as[0], s2hs[0], d2as[0], s2as[1], s2hs[1], d2as[1],
             s2as[2], s2hs[2], d2as[2], its[0], its[1], its[2])


def _final_body(t1_ref, t2_ref, t3_ref, cv_ref, out_ref):
    cv = cv_ref[...]
    acc = None
    for i, t_ref in enumerate((t1_ref, t2_ref, t3_ref)):
        t = t_ref[0] + t_ref[1]
        y = t[:, 1:2] / (t[:, 0:1] + EPS) + cv[0, i]
        term = y * cv[0, 3 + i]
        acc = term if acc is None else acc + term
    out_ref[...] = acc + cv[0, 6]


def _final(t2outs, cvec):
    blk3 = pl.BlockSpec((NC, BLK, 2), lambda i: (0, i, 0))
    return pl.pallas_call(
        _final_body,
        grid=(GRID,),
        in_specs=[blk3, blk3, blk3, pl.BlockSpec((1, 8), lambda i: (0, 0))],
        out_specs=pl.BlockSpec((BLK, 1), lambda i: (i, 0)),
        out_shape=jax.ShapeDtypeStruct((N, 1), jnp.float32),
    )(*t2outs, cvec)


def kernel(X, A1, A2, A3, edge_feature, params):
    p = params
    As, Ads, Ws, b1s, w2s, scs = [], [], [], [], [], []
    for i in (1, 2, 3):
        a_s = p[f"as1_{i}"].reshape(HEADS, H_FEATS)
        a_d = p[f"ad1_{i}"].reshape(HEADS, H_FEATS)
        zer = jnp.zeros((HEADS, HH), jnp.float32)
        rows = jnp.arange(HEADS)[:, None]
        cols = rows * H_FEATS + jnp.arange(H_FEATS)[None, :]
        As.append(zer.at[rows, cols].set(a_s).T)
        Ads.append(zer.at[rows, cols].set(a_d).T)
        Ws.append(p[f"W1_{i}"])
        b1s.append(p[f"b1_{i}"].reshape(1, HH))
        w2s.append(p[f"W2_{i}"])
        scs.append(jnp.stack([p[f"as2_{i}"].reshape(()), p[f"ad2_{i}"].reshape(())]).reshape(1, 2))
    Em = jnp.zeros((HEADS, HH), jnp.float32).at[
        jnp.arange(HEADS)[:, None],
        jnp.arange(HEADS)[:, None] * H_FEATS + jnp.arange(H_FEATS)[None, :],
    ].set(1.0)
    E2 = jnp.zeros((2, 32), jnp.float32).at[
        jnp.arange(2)[:, None],
        jnp.arange(2)[:, None] * H_FEATS + jnp.arange(H_FEATS)[None, :],
    ].set(1.0)

    s1 = _stage1(X, Ws, As, Ads, Em)
    accs, dens = [], []
    srcs, dsts = [], []
    for i, A in enumerate((A1, A2, A3)):
        H, Aq, D, IA, ID = s1[5 * i : 5 * i + 5]
        src = A[0].astype(jnp.int32)
        dst = A[1].astype(jnp.int32)
        srcs.append(src)
        dsts.append(dst)
        acc, den = _gat1_edges(
            src, dst,
            Aq.reshape(N * HEADS), D.reshape(N * HEADS), H.reshape(NC * N, 32),
            IA, ID.reshape(NC, 2 * N),
        )
        accs.append(acc)
        dens.append(den.reshape(NC, N, 2))

    s2 = _stage2(accs, dens, b1s, w2s, scs, E2)
    s2as = [s2[4 * i].reshape(N) for i in range(3)]
    s2hs = [s2[4 * i + 1].reshape(N) for i in range(3)]
    d2as = [s2[4 * i + 2].reshape(N) for i in range(3)]
    its = [s2[4 * i + 3].reshape(NC, 2 * N) for i in range(3)]

    t2outs = _gat2_edges(srcs, dsts, s2as, s2hs, d2as, its)

    cvec = jnp.concatenate(
        [
            jnp.stack([p["b2_1"][0], p["b2_2"][0], p["b2_3"][0]]),
            p["Wln"][:, 0],
            p["bln"],
            jnp.zeros((1,), jnp.float32),
        ]
    ).reshape(1, 8)
    return _final([t.reshape(NC, N, 2) for t in t2outs], cvec)
